# baseline (speedup 1.0000x reference)
import functools

import jax
import jax.numpy as jnp
from jax import lax
from jax.experimental import pallas as pl
from jax.experimental.pallas import tpu as pltpu
from jax.experimental.pallas import tpu_sc as plsc

NC, NS, L = 2, 16, 16
NW = NC * NS

BATCH = 16384
EMBED = 32
PACK = 128 // EMBED
NPACK = 1000000 // PACK
BPW = BATCH // NW
G = 64
NRND = BPW // G


def _sc_body(uidx_hbm, midx_hbm, ut2_hbm, mt2_hbm, out_hbm,
             uidx_v, midx_v, upk_v, mpk_v, ubuf0, ubuf1, mbuf0, mbuf1,
             out_v, sem):
    wid = lax.axis_index("s") * NC + lax.axis_index("c")
    base = pl.multiple_of(wid * BPW, BPW)

    pltpu.sync_copy(uidx_hbm.at[pl.ds(base, BPW)], uidx_v)
    pltpu.sync_copy(midx_hbm.at[pl.ds(base, BPW)], midx_v)

    iota = lax.broadcasted_iota(jnp.int32, (L,), 0)

    for c in range(BPW // L):
        sl = pl.ds(c * L, L)
        upk_v[sl] = jax.lax.shift_right_logical(uidx_v[sl], 2)
        mpk_v[sl] = jax.lax.shift_right_logical(midx_v[sl], 2)

    ubufs = (ubuf0, ubuf1)
    mbufs = (mbuf0, mbuf1)

    def fire(r, ubuf, mbuf):
        sl = pl.ds(pl.multiple_of(r * G, G), G)
        pltpu.async_copy(ut2_hbm.at[upk_v.at[sl]], ubuf, sem)
        pltpu.async_copy(mt2_hbm.at[mpk_v.at[sl]], mbuf, sem)

    def drain(ubuf, mbuf):
        pltpu.make_async_copy(ut2_hbm.at[pl.ds(0, G)], ubuf, sem).wait()
        pltpu.make_async_copy(mt2_hbm.at[pl.ds(0, G)], mbuf, sem).wait()

    def compute(r, ubuf, mbuf):
        for c in range(G // L):
            sl = pl.ds(pl.multiple_of(r * G + c * L, L), L)
            uoff = jax.lax.bitwise_and(uidx_v[sl], PACK - 1) * EMBED
            moff = jax.lax.bitwise_and(midx_v[sl], PACK - 1) * EMBED
            slot = iota + c * L
            acc = jnp.zeros((L,), jnp.float32)
            for d in range(EMBED):
                ucol = plsc.load_gather(ubuf, [slot, uoff + d])
                mcol = plsc.load_gather(mbuf, [slot, moff + d])
                acc = acc + ucol * mcol
            out_v[sl] = acc

    fire(0, ubufs[0], mbufs[0])

    def round2_body(h, _):
        r = h * 2

        @pl.when(r + 1 < NRND)
        def _f1():
            fire(r + 1, ubufs[1], mbufs[1])

        drain(ubufs[0], mbufs[0])
        compute(r, ubufs[0], mbufs[0])

        @pl.when(r + 2 < NRND)
        def _f2():
            fire(r + 2, ubufs[0], mbufs[0])

        @pl.when(r + 1 < NRND)
        def _c1():
            drain(ubufs[1], mbufs[1])
            compute(r + 1, ubufs[1], mbufs[1])

        return _

    lax.fori_loop(0, (NRND + 1) // 2, round2_body, 0)

    pltpu.sync_copy(out_v, out_hbm.at[pl.ds(base, BPW)])


@jax.jit
def _sc_call(uidx, midx, ut2, mt2):
    mesh = plsc.VectorSubcoreMesh(core_axis_name="c", subcore_axis_name="s")
    return pl.kernel(
        _sc_body,
        out_type=jax.ShapeDtypeStruct((BATCH,), jnp.float32),
        mesh=mesh,
        compiler_params=pltpu.CompilerParams(needs_layout_passes=False),
        scratch_types=[
            pltpu.VMEM((BPW,), jnp.int32),
            pltpu.VMEM((BPW,), jnp.int32),
            pltpu.VMEM((BPW,), jnp.int32),
            pltpu.VMEM((BPW,), jnp.int32),
            pltpu.VMEM((G, 128), jnp.float32),
            pltpu.VMEM((G, 128), jnp.float32),
            pltpu.VMEM((G, 128), jnp.float32),
            pltpu.VMEM((G, 128), jnp.float32),
            pltpu.VMEM((BPW,), jnp.float32),
            pltpu.SemaphoreType.DMA,
        ],
    )(uidx, midx, ut2, mt2)


def kernel(inputs, user_table, movie_table):
    uidx = inputs[:, 0]
    midx = inputs[:, 1]
    ut2 = user_table.reshape(NPACK, PACK * EMBED)
    mt2 = movie_table.reshape(NPACK, PACK * EMBED)
    out = _sc_call(uidx, midx, ut2, mt2)
    return out.reshape(BATCH, 1)


# speedup vs baseline: 1.4759x; 1.4759x over previous
import functools

import jax
import jax.numpy as jnp
from jax import lax
from jax.experimental import pallas as pl
from jax.experimental.pallas import tpu as pltpu
from jax.experimental.pallas import tpu_sc as plsc

NC, NS, L = 2, 16, 16
NW = NC * NS

BATCH = 16384
EMBED = 32
BPW = BATCH // NW
GRP = 128
NGRP = BPW // GRP
CHUNK = GRP // L


def _sc_body(uidx_hbm, midx_hbm, user_hbm, movie_hbm, out_hbm,
             uidx_v, midx_v, ubuf0, ubuf1, mbuf0, mbuf1, out_v, sem0, sem1):
    wid = lax.axis_index("s") * NC + lax.axis_index("c")
    base = pl.multiple_of(wid * BPW, BPW)

    pltpu.sync_copy(uidx_hbm.at[pl.ds(base, BPW)], uidx_v)
    pltpu.sync_copy(midx_hbm.at[pl.ds(base, BPW)], midx_v)

    iota = lax.broadcasted_iota(jnp.int32, (L,), 0)

    def fire(g, ubuf, mbuf, sem):
        gbase = pl.multiple_of(g * GRP, GRP)
        for c in range(CHUNK):
            uvec = uidx_v[pl.ds(gbase + c * L, L)]
            mvec = midx_v[pl.ds(gbase + c * L, L)]
            for j in range(L):
                r = c * L + j
                pltpu.async_copy(user_hbm.at[uvec[j]], ubuf.at[r], sem)
                pltpu.async_copy(movie_hbm.at[mvec[j]], mbuf.at[r], sem)

    def drain(ubuf, mbuf, sem):
        pltpu.make_async_copy(user_hbm.at[pl.ds(0, GRP), :], ubuf, sem).wait()
        pltpu.make_async_copy(user_hbm.at[pl.ds(0, GRP), :], mbuf, sem).wait()

    def compute(g, ubuf, mbuf):
        gbase = pl.multiple_of(g * GRP, GRP)
        for c in range(CHUNK):
            acc = jnp.zeros((L,), jnp.float32)
            for d in range(EMBED):
                dcol = jnp.full((L,), d, jnp.int32)
                rows = iota + c * L
                ucol = plsc.load_gather(ubuf, [rows, dcol])
                mcol = plsc.load_gather(mbuf, [rows, dcol])
                acc = acc + ucol * mcol
            out_v[pl.ds(pl.multiple_of(gbase + c * L, L), L)] = acc

    fire(0, ubuf0, mbuf0, sem0)

    def pair_body(h, _):
        g = h * 2

        @pl.when(g + 1 < NGRP)
        def _f1():
            fire(g + 1, ubuf1, mbuf1, sem1)

        drain(ubuf0, mbuf0, sem0)
        compute(g, ubuf0, mbuf0)

        @pl.when(g + 2 < NGRP)
        def _f2():
            fire(g + 2, ubuf0, mbuf0, sem0)

        @pl.when(g + 1 < NGRP)
        def _c1():
            drain(ubuf1, mbuf1, sem1)
            compute(g + 1, ubuf1, mbuf1)

        return _

    lax.fori_loop(0, (NGRP + 1) // 2, pair_body, 0)

    pltpu.sync_copy(out_v, out_hbm.at[pl.ds(base, BPW)])


@jax.jit
def _sc_call(uidx, midx, user_table, movie_table):
    mesh = plsc.VectorSubcoreMesh(core_axis_name="c", subcore_axis_name="s")
    return pl.kernel(
        _sc_body,
        out_type=jax.ShapeDtypeStruct((BATCH,), jnp.float32),
        mesh=mesh,
        compiler_params=pltpu.CompilerParams(needs_layout_passes=False,
                                             use_tc_tiling_on_sc=True),
        scratch_types=[
            pltpu.VMEM((BPW,), jnp.int32),
            pltpu.VMEM((BPW,), jnp.int32),
            pltpu.VMEM((GRP, EMBED), jnp.float32),
            pltpu.VMEM((GRP, EMBED), jnp.float32),
            pltpu.VMEM((GRP, EMBED), jnp.float32),
            pltpu.VMEM((GRP, EMBED), jnp.float32),
            pltpu.VMEM((BPW,), jnp.float32),
            pltpu.SemaphoreType.DMA,
            pltpu.SemaphoreType.DMA,
        ],
    )(uidx, midx, user_table, movie_table)


def kernel(inputs, user_table, movie_table):
    uidx = inputs[:, 0]
    midx = inputs[:, 1]
    out = _sc_call(uidx, midx, user_table, movie_table)
    return out.reshape(BATCH, 1)
